# baseline (speedup 1.0000x reference)
import functools

import jax
import jax.numpy as jnp
from jax import lax
from jax.experimental import pallas as pl
from jax.experimental.pallas import tpu as pltpu
from jax.experimental.pallas import tpu_sc as plsc

_LN2 = 0.6931471805599453
_SQRT2 = 1.4142135623730951


def _ln(x):
    bits = plsc.bitcast(x, jnp.int32)
    e = (bits >> 23) - 127
    mbits = (bits & 0x007FFFFF) | 0x3F800000
    m = plsc.bitcast(mbits, jnp.float32)
    big = m > _SQRT2
    m = jnp.where(big, m * 0.5, m)
    e = jnp.where(big, e + 1, e)
    t = (m - 1.0) / (m + 1.0)
    t2 = t * t
    p = jnp.float32(1.0 / 9.0) + t2 * 0.0
    p = 1.0 / 7.0 + t2 * p
    p = 1.0 / 5.0 + t2 * p
    p = 1.0 / 3.0 + t2 * p
    p = 1.0 + t2 * p
    return e.astype(jnp.float32) * _LN2 + 2.0 * t * p


def kernel(feat, taken_actions, policy):
    B = feat.shape[0]
    A = policy.shape[1]
    NW = 32
    b_per_w = B // NW
    n_chunks = 4
    c_rows = b_per_w // n_chunks
    n_groups = c_rows // 16

    mesh = plsc.VectorSubcoreMesh(core_axis_name="c", subcore_axis_name="s")

    @functools.partial(
        pl.kernel,
        mesh=mesh,
        out_type=jax.ShapeDtypeStruct((B,), jnp.float32),
        compiler_params=pltpu.CompilerParams(
            needs_layout_passes=False, use_tc_tiling_on_sc=False),
        scratch_types=[
            pltpu.VMEM((b_per_w,), jnp.int32),
            pltpu.VMEM((b_per_w,), jnp.int32),
            pltpu.VMEM((b_per_w, A), jnp.float32),
            pltpu.VMEM((b_per_w,), jnp.float32),
            pltpu.SemaphoreType.DMA,
            pltpu.SemaphoreType.DMA,
            pltpu.SemaphoreType.DMA,
            pltpu.SemaphoreType.DMA,
        ],
    )
    def sc_kernel(feat_hbm, act_hbm, table_hbm, out_hbm,
                  idx_v, act_v, rows_v, out_v, s0, s1, s2, s3):
        sems = [s0, s1, s2, s3]
        wid = lax.axis_index("s") * 2 + lax.axis_index("c")
        base = wid * b_per_w
        pltpu.sync_copy(feat_hbm.at[pl.ds(base, b_per_w)], idx_v)
        pltpu.sync_copy(act_hbm.at[pl.ds(base, b_per_w)], act_v)

        copies = []
        for c in range(n_chunks):
            copies.append(pltpu.async_copy(
                table_hbm.at[idx_v.at[pl.ds(c * c_rows, c_rows)]],
                rows_v.at[pl.ds(c * c_rows, c_rows)],
                sems[c]))

        lane = lax.iota(jnp.int32, 16)
        for c in range(n_chunks):
            copies[c].wait()
            for g in range(n_groups):
                off = c * c_rows + g * 16
                row_ids = lane + off
                acts = act_v[pl.ds(off, 16)]

                def max_body(j, m):
                    col = jnp.full((16,), 0, jnp.int32) + j
                    v = plsc.load_gather(rows_v, [row_ids, col])
                    return jnp.maximum(m, v)

                m = plsc.load_gather(
                    rows_v, [row_ids, jnp.zeros((16,), jnp.int32)])
                m = lax.fori_loop(1, A, max_body, m)

                def sum_body(j, s):
                    col = jnp.full((16,), 0, jnp.int32) + j
                    v = plsc.load_gather(rows_v, [row_ids, col])
                    return s + jnp.exp(v - m)

                s = lax.fori_loop(0, A, sum_body,
                                  jnp.zeros((16,), jnp.float32))

                la = plsc.load_gather(rows_v, [row_ids, acts])
                out_v[pl.ds(off, 16)] = la - m - _ln(s)

        pltpu.sync_copy(out_v, out_hbm.at[pl.ds(base, b_per_w)])

    return sc_kernel(feat, taken_actions, policy)


# speedup vs baseline: 1.0503x; 1.0503x over previous
import functools

import jax
import jax.numpy as jnp
from jax import lax
from jax.experimental import pallas as pl
from jax.experimental.pallas import tpu as pltpu
from jax.experimental.pallas import tpu_sc as plsc

_LN2 = 0.6931471805599453
_SQRT2 = 1.4142135623730951


def _ln(x):
    bits = plsc.bitcast(x, jnp.int32)
    e = (bits >> 23) - 127
    mbits = (bits & 0x007FFFFF) | 0x3F800000
    m = plsc.bitcast(mbits, jnp.float32)
    big = m > _SQRT2
    m = jnp.where(big, m * 0.5, m)
    e = jnp.where(big, e + 1, e)
    t = (m - 1.0) / (m + 1.0)
    t2 = t * t
    p = jnp.float32(1.0 / 9.0) + t2 * 0.0
    p = 1.0 / 7.0 + t2 * p
    p = 1.0 / 5.0 + t2 * p
    p = 1.0 / 3.0 + t2 * p
    p = 1.0 + t2 * p
    return e.astype(jnp.float32) * _LN2 + 2.0 * t * p


def kernel(feat, taken_actions, policy):
    B = feat.shape[0]
    A = policy.shape[1]
    NW = 32
    b_per_w = B // NW
    n_chunks = 4
    c_rows = b_per_w // n_chunks
    n_groups = c_rows // 16

    mesh = plsc.VectorSubcoreMesh(core_axis_name="c", subcore_axis_name="s")

    @functools.partial(
        pl.kernel,
        mesh=mesh,
        out_type=jax.ShapeDtypeStruct((B,), jnp.float32),
        compiler_params=pltpu.CompilerParams(
            needs_layout_passes=False, use_tc_tiling_on_sc=False),
        scratch_types=[
            pltpu.VMEM((b_per_w,), jnp.int32),
            pltpu.VMEM((b_per_w,), jnp.int32),
            pltpu.VMEM((b_per_w, A), jnp.float32),
            pltpu.VMEM((b_per_w,), jnp.float32),
            pltpu.SemaphoreType.DMA,
            pltpu.SemaphoreType.DMA,
            pltpu.SemaphoreType.DMA,
            pltpu.SemaphoreType.DMA,
        ],
    )
    def sc_kernel(feat_hbm, act_hbm, table_hbm, out_hbm,
                  idx_v, act_v, rows_v, out_v, s0, s1, s2, s3):
        sems = [s0, s1, s2, s3]
        wid = lax.axis_index("s") * 2 + lax.axis_index("c")
        base = wid * b_per_w
        pltpu.sync_copy(feat_hbm.at[pl.ds(base, b_per_w)], idx_v)
        pltpu.sync_copy(act_hbm.at[pl.ds(base, b_per_w)], act_v)

        copies = []
        for c in range(n_chunks):
            copies.append(pltpu.async_copy(
                table_hbm.at[idx_v.at[pl.ds(c * c_rows, c_rows)]],
                rows_v.at[pl.ds(c * c_rows, c_rows)],
                sems[c]))

        lane = lax.iota(jnp.int32, 16)
        cols = [jnp.full((16,), j, jnp.int32) for j in range(A)]

        for c in range(n_chunks):
            copies[c].wait()

            def group_body(g, carry, c=c):
                off = c * c_rows + g * 16
                row_ids = lane + off
                acts = act_v[pl.ds(off, 16)]

                vs = [plsc.load_gather(rows_v, [row_ids, cols[j]])
                      for j in range(4)]
                ms = vs
                for j in range(4, A, 4):
                    for k in range(4):
                        v = plsc.load_gather(rows_v, [row_ids, cols[j + k]])
                        ms[k] = jnp.maximum(ms[k], v)
                m = jnp.maximum(jnp.maximum(ms[0], ms[1]),
                                jnp.maximum(ms[2], ms[3]))

                ss = [jnp.zeros((16,), jnp.float32) for _ in range(4)]
                for j in range(0, A, 4):
                    for k in range(4):
                        v = plsc.load_gather(rows_v, [row_ids, cols[j + k]])
                        ss[k] = ss[k] + jnp.exp(v - m)
                s = (ss[0] + ss[1]) + (ss[2] + ss[3])

                la = plsc.load_gather(rows_v, [row_ids, acts])
                out_v[pl.ds(off, 16)] = la - m - _ln(s)
                return carry

            lax.fori_loop(0, n_groups, group_body, 0)

        pltpu.sync_copy(out_v, out_hbm.at[pl.ds(base, b_per_w)])

    return sc_kernel(feat, taken_actions, policy)
